# Initial kernel scaffold; baseline (speedup 1.0000x reference)
import jax
import jax.numpy as jnp
from jax.experimental import pallas as pl


def kernel(thax, pars, rules, pos_vals, neg_vals, init_table, W, b, w_eval, b_eval):
    raise NotImplementedError("write your pallas kernel here")


# speedup vs baseline: 4.8433x; 4.8433x over previous
import jax
import jax.numpy as jnp
from jax import lax
from jax.experimental import pallas as pl
from jax.experimental.pallas import tpu as pltpu
from jax.experimental.pallas import tpu_sc as plsc

N_INIT = 16384
N_LAYERS = 8
LAYER = 2048
D = 128
N_RULES = 32
N_TOTAL = N_INIT + N_LAYERS * LAYER

_NC, _NS = 2, 16
_NW = _NC * _NS
_CHUNK = 128

_BLK = 256


def _sc_gather(table, idx, out_rows):
    B = idx.shape[0]
    n_chunks = B // (_NW * _CHUNK)
    assert n_chunks * _NW * _CHUNK == B
    mesh = plsc.VectorSubcoreMesh(core_axis_name="c", subcore_axis_name="s",
                                  num_cores=_NC, num_subcores=_NS)

    def body(table_hbm, idx_hbm, out_hbm, idx_v, rows_v, sem):
        wid = lax.axis_index("s") * _NC + lax.axis_index("c")
        for c in range(n_chunks):
            base = (wid * n_chunks + c) * _CHUNK
            pltpu.sync_copy(idx_hbm.at[pl.ds(base, _CHUNK)], idx_v)
            pltpu.async_copy(table_hbm.at[idx_v], rows_v, sem).wait()
            pltpu.sync_copy(rows_v, out_hbm.at[pl.ds(base, _CHUNK)])

    fn = pl.kernel(
        body,
        out_type=jax.ShapeDtypeStruct((out_rows, D), jnp.float32),
        mesh=mesh,
        scratch_types=[pltpu.VMEM((_CHUNK,), jnp.int32),
                       pltpu.VMEM((_CHUNK, D), jnp.float32),
                       pltpu.SemaphoreType.DMA],
    )
    return fn(table, idx)


def _layer_body(x_ref, r_ref, wt_ref, b_ref, h_ref):
    x = x_ref[...]
    rule = r_ref[...]
    mask = (rule == lax.broadcasted_iota(jnp.int32, (_BLK, N_RULES), 1))
    maskf = mask.astype(jnp.float32)
    y = jnp.dot(x, wt_ref[...], preferred_element_type=jnp.float32)
    acc = jnp.zeros((_BLK, D), jnp.float32)
    for r in range(N_RULES):
        acc = acc + maskf[:, r:r + 1] * (y[:, r * D:(r + 1) * D]
                                         + b_ref[r:r + 1, :])
    h_ref[...] = jnp.tanh(acc)


def _tc_layer(x, rules_col, w_t, b):
    grid = (LAYER // _BLK,)
    return pl.pallas_call(
        _layer_body,
        grid=grid,
        in_specs=[
            pl.BlockSpec((_BLK, 2 * D), lambda i: (i, 0)),
            pl.BlockSpec((_BLK, 1), lambda i: (i, 0)),
            pl.BlockSpec((2 * D, N_RULES * D), lambda i: (0, 0)),
            pl.BlockSpec((N_RULES, D), lambda i: (0, 0)),
        ],
        out_specs=pl.BlockSpec((_BLK, D), lambda i: (i, 0)),
        out_shape=jax.ShapeDtypeStruct((LAYER, D), jnp.float32),
    )(x, rules_col, w_t, b)


def _eval_body(s_ref, wbig_ref, be_ref, pos_ref, neg_ref,
               loss_ref, pr_ref, nr_ref):
    logits = jnp.dot(s_ref[...], wbig_ref[...],
                     preferred_element_type=jnp.float32) + be_ref[0, 0]
    pos = pos_ref[...]
    neg = neg_ref[...]
    tot = pos + neg
    t = pos / tot
    tot_pos = jnp.sum(pos)
    tot_neg = jnp.sum(neg)
    pw = tot_neg / tot_pos
    common = jnp.log1p(jnp.exp(-jnp.abs(logits)))
    sp_pos = jnp.maximum(logits, 0.0) + common
    sp_neg = jnp.maximum(-logits, 0.0) + common
    per = pw * t * sp_neg + (1.0 - t) * sp_pos
    loss_ref[0, 0] = jnp.sum(tot * per)
    pr_ref[0, 0] = jnp.sum(jnp.where(logits >= 0.0, pos, 0.0)) / tot_pos
    nr_ref[0, 0] = jnp.sum(jnp.where(logits < 0.0, neg, 0.0)) / tot_neg


def _tc_eval(store_flat, w_big, b_eval_2d, pos2d, neg2d):
    out = pl.pallas_call(
        _eval_body,
        in_specs=[
            pl.BlockSpec(memory_space=pltpu.VMEM),
            pl.BlockSpec(memory_space=pltpu.VMEM),
            pl.BlockSpec(memory_space=pltpu.SMEM),
            pl.BlockSpec(memory_space=pltpu.VMEM),
            pl.BlockSpec(memory_space=pltpu.VMEM),
        ],
        out_specs=[
            pl.BlockSpec(memory_space=pltpu.SMEM),
            pl.BlockSpec(memory_space=pltpu.SMEM),
            pl.BlockSpec(memory_space=pltpu.SMEM),
        ],
        out_shape=[
            jax.ShapeDtypeStruct((1, 1), jnp.float32),
            jax.ShapeDtypeStruct((1, 1), jnp.float32),
            jax.ShapeDtypeStruct((1, 1), jnp.float32),
        ],
    )(store_flat, w_big, b_eval_2d, pos2d, neg2d)
    return out


def kernel(thax, pars, rules, pos_vals, neg_vals, init_table, W, b,
           w_eval, b_eval):
    thax = thax.astype(jnp.int32)
    store = _sc_gather(init_table, thax, N_TOTAL)

    w_t = jnp.transpose(W, (1, 0, 2)).reshape(2 * D, N_RULES * D)
    rules2 = rules.astype(jnp.int32).reshape(N_LAYERS, LAYER)

    for l in range(N_LAYERS):
        par_idx = pars[l].reshape(-1).astype(jnp.int32)
        x = _sc_gather(store, par_idx, 2 * LAYER)
        x2 = x.reshape(LAYER, 2 * D)
        h = _tc_layer(x2, rules2[l].reshape(LAYER, 1), w_t, b)
        store = lax.dynamic_update_slice(store, h, (N_INIT + l * LAYER, 0))

    eye = jnp.eye(D, dtype=jnp.float32)
    w_big = (eye[:, None, :] * w_eval[None, :, None]).reshape(D * D, D)
    store_flat = store.reshape(N_TOTAL // D, D * D)
    pos2d = pos_vals.reshape(N_TOTAL // D, D)
    neg2d = neg_vals.reshape(N_TOTAL // D, D)
    loss, pr, nr = _tc_eval(store_flat, w_big,
                            b_eval.reshape(1, 1).astype(jnp.float32),
                            pos2d, neg2d)
    return (loss.reshape(1), pr.reshape(()), nr.reshape(()))


# speedup vs baseline: 5.4050x; 1.1160x over previous
import jax
import jax.numpy as jnp
from jax import lax
from jax.experimental import pallas as pl
from jax.experimental.pallas import tpu as pltpu
from jax.experimental.pallas import tpu_sc as plsc

N_INIT = 16384
N_LAYERS = 8
LAYER = 2048
D = 128
N_RULES = 32
N_TOTAL = N_INIT + N_LAYERS * LAYER

_NC, _NS = 2, 16
_NW = _NC * _NS
_CHUNK = 128

_BLK = 256


def _sc_gather(table, idx, out_rows):
    B = idx.shape[0]
    n_chunks = B // (_NW * _CHUNK)
    assert n_chunks * _NW * _CHUNK == B
    mesh = plsc.VectorSubcoreMesh(core_axis_name="c", subcore_axis_name="s",
                                  num_cores=_NC, num_subcores=_NS)

    def body(table_hbm, idx_hbm, out_hbm, idx_v, rows_v, sem):
        wid = lax.axis_index("s") * _NC + lax.axis_index("c")
        for c in range(n_chunks):
            base = (wid * n_chunks + c) * _CHUNK
            pltpu.sync_copy(idx_hbm.at[pl.ds(base, _CHUNK)], idx_v)
            pltpu.async_copy(table_hbm.at[idx_v], rows_v, sem).wait()
            pltpu.sync_copy(rows_v, out_hbm.at[pl.ds(base, _CHUNK)])

    fn = pl.kernel(
        body,
        out_type=jax.ShapeDtypeStruct((out_rows, D), jnp.float32),
        mesh=mesh,
        scratch_types=[pltpu.VMEM((_CHUNK,), jnp.int32),
                       pltpu.VMEM((_CHUNK, D), jnp.float32),
                       pltpu.SemaphoreType.DMA],
    )
    return fn(table, idx)


def _layer_body(store_ref, x_ref, r_ref, wt_ref, b_ref, h_ref):
    del store_ref
    x = x_ref[...].astype(jnp.bfloat16)
    rule = r_ref[...]
    mask = (rule == lax.broadcasted_iota(jnp.int32, (_BLK, N_RULES), 1))
    maskf = mask.astype(jnp.float32)
    y = jnp.dot(x, wt_ref[...], preferred_element_type=jnp.float32)
    acc = jnp.zeros((_BLK, D), jnp.float32)
    for r in range(N_RULES):
        acc = acc + maskf[:, r:r + 1] * (y[:, r * D:(r + 1) * D]
                                         + b_ref[r:r + 1, :])
    h_ref[...] = jnp.tanh(acc)


def _tc_layer(store, x, rules_col, w_t, b, layer_idx):
    grid = (LAYER // _BLK,)
    row_blk0 = (N_INIT + layer_idx * LAYER) // _BLK
    return pl.pallas_call(
        _layer_body,
        grid=grid,
        in_specs=[
            pl.BlockSpec(memory_space=pl.ANY),
            pl.BlockSpec((_BLK, 2 * D), lambda i: (i, 0)),
            pl.BlockSpec((_BLK, 1), lambda i: (i, 0)),
            pl.BlockSpec((2 * D, N_RULES * D), lambda i: (0, 0)),
            pl.BlockSpec((N_RULES, D), lambda i: (0, 0)),
        ],
        out_specs=pl.BlockSpec((_BLK, D), lambda i: (row_blk0 + i, 0)),
        out_shape=jax.ShapeDtypeStruct((N_TOTAL, D), jnp.float32),
        input_output_aliases={0: 0},
    )(store, x, rules_col, w_t, b)


def _eval_body(s_ref, wbig_ref, be_ref, pos_ref, neg_ref,
               loss_ref, pr_ref, nr_ref):
    logits = jnp.dot(s_ref[...], wbig_ref[...],
                     preferred_element_type=jnp.float32) + be_ref[0, 0]
    pos = pos_ref[...]
    neg = neg_ref[...]
    tot = pos + neg
    t = pos / tot
    tot_pos = jnp.sum(pos)
    tot_neg = jnp.sum(neg)
    pw = tot_neg / tot_pos
    common = jnp.log1p(jnp.exp(-jnp.abs(logits)))
    sp_pos = jnp.maximum(logits, 0.0) + common
    sp_neg = jnp.maximum(-logits, 0.0) + common
    per = pw * t * sp_neg + (1.0 - t) * sp_pos
    loss_ref[0, 0] = jnp.sum(tot * per)
    pr_ref[0, 0] = jnp.sum(jnp.where(logits >= 0.0, pos, 0.0)) / tot_pos
    nr_ref[0, 0] = jnp.sum(jnp.where(logits < 0.0, neg, 0.0)) / tot_neg


def _tc_eval(store_flat, w_big, b_eval_2d, pos2d, neg2d):
    return pl.pallas_call(
        _eval_body,
        in_specs=[
            pl.BlockSpec(memory_space=pltpu.VMEM),
            pl.BlockSpec(memory_space=pltpu.VMEM),
            pl.BlockSpec(memory_space=pltpu.SMEM),
            pl.BlockSpec(memory_space=pltpu.VMEM),
            pl.BlockSpec(memory_space=pltpu.VMEM),
        ],
        out_specs=[
            pl.BlockSpec(memory_space=pltpu.SMEM),
            pl.BlockSpec(memory_space=pltpu.SMEM),
            pl.BlockSpec(memory_space=pltpu.SMEM),
        ],
        out_shape=[
            jax.ShapeDtypeStruct((1, 1), jnp.float32),
            jax.ShapeDtypeStruct((1, 1), jnp.float32),
            jax.ShapeDtypeStruct((1, 1), jnp.float32),
        ],
    )(store_flat, w_big, b_eval_2d, pos2d, neg2d)


def kernel(thax, pars, rules, pos_vals, neg_vals, init_table, W, b,
           w_eval, b_eval):
    thax = thax.astype(jnp.int32)
    store = _sc_gather(init_table, thax, N_TOTAL)

    w_t = jnp.transpose(W, (1, 0, 2)).reshape(2 * D, N_RULES * D)
    w_t = w_t.astype(jnp.bfloat16)
    rules2 = rules.astype(jnp.int32).reshape(N_LAYERS, LAYER)

    for l in range(N_LAYERS):
        par_idx = pars[l].reshape(-1).astype(jnp.int32)
        x = _sc_gather(store, par_idx, 2 * LAYER)
        x2 = x.reshape(LAYER, 2 * D)
        store = _tc_layer(store, x2, rules2[l].reshape(LAYER, 1), w_t, b, l)

    eye = jnp.eye(D, dtype=jnp.float32)
    w_big = (eye[:, None, :] * w_eval[None, :, None]).reshape(D * D, D)
    store_flat = store.reshape(N_TOTAL // D, D * D)
    pos2d = pos_vals.reshape(N_TOTAL // D, D)
    neg2d = neg_vals.reshape(N_TOTAL // D, D)
    loss, pr, nr = _tc_eval(store_flat, w_big,
                            b_eval.reshape(1, 1).astype(jnp.float32),
                            pos2d, neg2d)
    return (loss.reshape(1), pr.reshape(()), nr.reshape(()))


# speedup vs baseline: 6.0854x; 1.1259x over previous
import jax
import jax.numpy as jnp
from jax import lax
from jax.experimental import pallas as pl
from jax.experimental.pallas import tpu as pltpu
from jax.experimental.pallas import tpu_sc as plsc

N_INIT = 16384
N_LAYERS = 8
LAYER = 2048
D = 128
N_RULES = 32
N_TOTAL = N_INIT + N_LAYERS * LAYER

_NC, _NS = 2, 16
_NW = _NC * _NS
_CHUNK = 128

_BLK = 256


def _sc_gather(table, idx, out_rows, n_gather, idx_off):
    n_chunks = n_gather // (_NW * _CHUNK)
    assert n_chunks * _NW * _CHUNK == n_gather
    mesh = plsc.VectorSubcoreMesh(core_axis_name="c", subcore_axis_name="s",
                                  num_cores=_NC, num_subcores=_NS)

    def body(table_hbm, idx_hbm, out_hbm, idx_v, rows_v, sem):
        wid = lax.axis_index("s") * _NC + lax.axis_index("c")
        for c in range(n_chunks):
            base = (wid * n_chunks + c) * _CHUNK
            pltpu.sync_copy(idx_hbm.at[pl.ds(idx_off + base, _CHUNK)], idx_v)
            pltpu.async_copy(table_hbm.at[idx_v], rows_v, sem).wait()
            pltpu.sync_copy(rows_v, out_hbm.at[pl.ds(base, _CHUNK)])

    fn = pl.kernel(
        body,
        out_type=jax.ShapeDtypeStruct((out_rows, D), jnp.float32),
        mesh=mesh,
        scratch_types=[pltpu.VMEM((_CHUNK,), jnp.int32),
                       pltpu.VMEM((_CHUNK, D), jnp.float32),
                       pltpu.SemaphoreType.DMA],
    )
    return fn(table, idx)


def _layer_body(store_ref, xa_ref, xb_ref, r_ref, wt_ref, b_ref, h_ref):
    del store_ref
    xa = xa_ref[...].astype(jnp.bfloat16)
    xb = xb_ref[...].astype(jnp.bfloat16)
    x = jnp.concatenate([xa, xb], axis=1)
    rule = r_ref[0]
    mask = (rule == lax.broadcasted_iota(jnp.int32, (_BLK, N_RULES), 1))
    maskf = mask.astype(jnp.float32)
    y = jnp.dot(x, wt_ref[...], preferred_element_type=jnp.float32)
    acc = jnp.zeros((_BLK, D), jnp.float32)
    for r in range(N_RULES):
        acc = acc + maskf[:, r:r + 1] * (y[:, r * D:(r + 1) * D]
                                         + b_ref[r:r + 1, :])
    h_ref[...] = jnp.tanh(acc)


def _tc_layer(store, x, rules2, w_t, b, layer_idx):
    grid = (LAYER // _BLK,)
    row_blk0 = (N_INIT + layer_idx * LAYER) // _BLK
    par1_blk0 = LAYER // _BLK
    return pl.pallas_call(
        _layer_body,
        grid=grid,
        in_specs=[
            pl.BlockSpec(memory_space=pl.ANY),
            pl.BlockSpec((_BLK, D), lambda i: (i, 0)),
            pl.BlockSpec((_BLK, D), lambda i: (par1_blk0 + i, 0)),
            pl.BlockSpec((1, _BLK, 1), lambda i: (layer_idx, i, 0)),
            pl.BlockSpec((2 * D, N_RULES * D), lambda i: (0, 0)),
            pl.BlockSpec((N_RULES, D), lambda i: (0, 0)),
        ],
        out_specs=pl.BlockSpec((_BLK, D), lambda i: (row_blk0 + i, 0)),
        out_shape=jax.ShapeDtypeStruct((N_TOTAL, D), jnp.float32),
        input_output_aliases={0: 0},
    )(store, x, x, rules2, w_t, b)


def _eval_body(s_ref, wbig_ref, be_ref, pos_ref, neg_ref,
               loss_ref, pr_ref, nr_ref):
    logits = jnp.dot(s_ref[...].astype(jnp.bfloat16), wbig_ref[...],
                     preferred_element_type=jnp.float32) + be_ref[0, 0]
    pos = pos_ref[...]
    neg = neg_ref[...]
    tot = pos + neg
    t = pos / tot
    tot_pos = jnp.sum(pos)
    tot_neg = jnp.sum(neg)
    pw = tot_neg / tot_pos
    common = jnp.log1p(jnp.exp(-jnp.abs(logits)))
    sp_pos = jnp.maximum(logits, 0.0) + common
    sp_neg = jnp.maximum(-logits, 0.0) + common
    per = pw * t * sp_neg + (1.0 - t) * sp_pos
    loss_ref[0, 0] = jnp.sum(tot * per)
    pr_ref[0, 0] = jnp.sum(jnp.where(logits >= 0.0, pos, 0.0)) / tot_pos
    nr_ref[0, 0] = jnp.sum(jnp.where(logits < 0.0, neg, 0.0)) / tot_neg


def _tc_eval(store_flat, w_big, b_eval_2d, pos2d, neg2d):
    return pl.pallas_call(
        _eval_body,
        in_specs=[
            pl.BlockSpec(memory_space=pltpu.VMEM),
            pl.BlockSpec(memory_space=pltpu.VMEM),
            pl.BlockSpec(memory_space=pltpu.SMEM),
            pl.BlockSpec(memory_space=pltpu.VMEM),
            pl.BlockSpec(memory_space=pltpu.VMEM),
        ],
        out_specs=[
            pl.BlockSpec(memory_space=pltpu.SMEM),
            pl.BlockSpec(memory_space=pltpu.SMEM),
            pl.BlockSpec(memory_space=pltpu.SMEM),
        ],
        out_shape=[
            jax.ShapeDtypeStruct((1, 1), jnp.float32),
            jax.ShapeDtypeStruct((1, 1), jnp.float32),
            jax.ShapeDtypeStruct((1, 1), jnp.float32),
        ],
    )(store_flat, w_big, b_eval_2d, pos2d, neg2d)


def kernel(thax, pars, rules, pos_vals, neg_vals, init_table, W, b,
           w_eval, b_eval):
    thax = thax.astype(jnp.int32)
    store = _sc_gather(init_table, thax, N_TOTAL, N_INIT, 0)

    w_t = jnp.transpose(W, (1, 0, 2)).astype(jnp.bfloat16)
    w_t = w_t.reshape(2 * D, N_RULES * D)
    rules2 = rules.astype(jnp.int32).reshape(N_LAYERS, LAYER, 1)
    parsT = pars.astype(jnp.int32).transpose(0, 2, 1).reshape(
        N_LAYERS * 2 * LAYER)

    for l in range(N_LAYERS):
        x = _sc_gather(store, parsT, 2 * LAYER, 2 * LAYER, l * 2 * LAYER)
        store = _tc_layer(store, x, rules2, w_t, b, l)

    eye = jnp.eye(D, dtype=jnp.float32)
    w_big = (eye[:, None, :] * w_eval[None, :, None]).reshape(D * D, D)
    w_big = w_big.astype(jnp.bfloat16)
    store_flat = store.reshape(N_TOTAL // D, D * D)
    pos2d = pos_vals.reshape(N_TOTAL // D, D)
    neg2d = neg_vals.reshape(N_TOTAL // D, D)
    loss, pr, nr = _tc_eval(store_flat, w_big,
                            b_eval.reshape(1, 1).astype(jnp.float32),
                            pos2d, neg2d)
    return (loss.reshape(1), pr.reshape(()), nr.reshape(()))
